# baseline (speedup 1.0000x reference)
import functools

import jax
import jax.numpy as jnp
from jax import lax
from jax.experimental import pallas as pl
from jax.experimental.pallas import tpu as pltpu
from jax.experimental.pallas import tpu_sc as plsc

T, D, F, E, TOP_K = 2048, 1024, 1024, 8, 2
LIMIT = 7.0
ALPHA = 1.702
EPS = 1e-5

BP = 128
NPAD = 5120
NTILES = NPAD // BP

SC_NC, SC_NS = 2, 16
HALF = NPAD // SC_NC
SPT = HALF // SC_NS
PPT = (2 * T) // SC_NS
NALLOC = NPAD + 16
GCH = 40
TPT = T // (SC_NC * SC_NS)
CCH = 16


def _deint_body(p_ref, wg_ref, wu_ref):
    v = p_ref[0]
    wg_ref[0] = jax.lax.bitcast_convert_type(
        v << 16, jnp.float32).astype(jnp.bfloat16)
    wu_ref[0] = jax.lax.bitcast_convert_type(
        v & jnp.int32(-65536), jnp.float32).astype(jnp.bfloat16)


def _router_body(x_ref, nw_ref, rwt_ref, rb_ref,
                 t_ref, dest_ref, wp_ref, te_ref):
    xx = x_ref[...]
    ms = jnp.mean(xx * xx, axis=-1, keepdims=True)
    t = xx * jax.lax.rsqrt(ms + EPS) * nw_ref[...]
    t_ref[...] = t
    g = jnp.dot(t, rwt_ref[...], preferred_element_type=jnp.float32)
    g = g + rb_ref[...]
    m = jnp.max(g, axis=-1, keepdims=True)
    eg = jnp.exp(g - m)
    p = eg / jnp.sum(eg, axis=-1, keepdims=True)
    eidx = jax.lax.broadcasted_iota(jnp.int32, p.shape, 1)
    v1 = jnp.max(p, axis=-1, keepdims=True)
    i1 = jnp.min(jnp.where(p >= v1, eidx, E), axis=-1, keepdims=True)
    p2 = jnp.where(eidx == i1, -jnp.inf, p)
    v2 = jnp.max(p2, axis=-1, keepdims=True)
    i2 = jnp.min(jnp.where(p2 >= v2, eidx, E), axis=-1, keepdims=True)
    s = v1 + v2
    wp_ref[...] = jnp.concatenate([v1 / s, v2 / s], axis=1)

    oh = ((eidx == i1) | (eidx == i2)).astype(jnp.bfloat16)
    ri = jax.lax.broadcasted_iota(jnp.int32, (T, T), 0)
    ci = jax.lax.broadcasted_iota(jnp.int32, (T, T), 1)
    tri = (ri > ci).astype(jnp.bfloat16)
    acc = jnp.dot(tri, oh, preferred_element_type=jnp.float32)
    cnt = jnp.sum(oh.astype(jnp.float32), axis=0, keepdims=True)
    cp = jnp.ceil(cnt / BP) * BP
    ei8 = jax.lax.broadcasted_iota(jnp.int32, (E, E), 0)
    ej8 = jax.lax.broadcasted_iota(jnp.int32, (E, E), 1)
    upper = (ei8 < ej8).astype(jnp.float32)
    starts = jnp.dot(cp, upper, preferred_element_type=jnp.float32)
    total = jnp.sum(cp, axis=-1, keepdims=True)

    def sel(ik, mat):
        return jnp.sum(jnp.where(eidx == ik, mat, 0.0), axis=-1,
                       keepdims=True)

    starts_b = jnp.broadcast_to(starts, (T, E))
    d0 = sel(i1, starts_b) + sel(i1, acc)
    d1 = sel(i2, starts_b) + sel(i2, acc)
    dest_ref[...] = jnp.concatenate([d0, d1], axis=1).astype(jnp.int32)

    pos = (jax.lax.broadcasted_iota(jnp.int32, (64, E), 0) * BP).astype(
        jnp.float32)
    n_le = jnp.sum((jnp.broadcast_to(starts, (64, E)) <= pos).astype(
        jnp.int32), axis=-1, keepdims=True)
    tile_e = n_le - 1
    te_ref[...] = jnp.where(pos[:, :1] < total, tile_e, -1)


def _scatter_body(dest_hbm, wflat_hbm, stid_hbm, sw_hbm,
                  zi_v, zf_v, didx_v, tid_v, wv_v):
    c = lax.axis_index("c")
    s = lax.axis_index("s")
    half = c * HALF
    my_slot0 = half + s * SPT

    for i in range(SPT // 16):
        zi_v[pl.ds(i * 16, 16)] = jnp.zeros((16,), jnp.int32)
        zf_v[pl.ds(i * 16, 16)] = jnp.zeros((16,), jnp.float32)
    pltpu.sync_copy(zi_v, stid_hbm.at[pl.ds(my_slot0, SPT)])
    pltpu.sync_copy(zf_v, sw_hbm.at[pl.ds(my_slot0, SPT)])
    plsc.subcore_barrier()

    p0 = s * PPT
    pltpu.sync_copy(dest_hbm.at[pl.ds(s * (PPT // 128), PPT // 128)],
                    didx_v)
    pltpu.sync_copy(wflat_hbm.at[pl.ds(s * (PPT // 128), PPT // 128)],
                    wv_v)
    dump = jnp.int32(NPAD) + 8 * c
    for j in range(PPT // 128):
        for i in range(128 // 16):
            d = didx_v[j, pl.ds(i * 16, 16)]
            keep = (d >= half) & (d < half + HALF)
            didx_v[j, pl.ds(i * 16, 16)] = jnp.where(keep, d, dump)
            pid = lax.iota(jnp.int32, 16) + (p0 + j * 128 + i * 16)
            tid_v[j, pl.ds(i * 16, 16)] = pid >> 1

    for j in range(PPT // 128):
        pltpu.sync_copy(tid_v.at[j], stid_hbm.at[didx_v.at[j]])
        pltpu.sync_copy(wv_v.at[j], sw_hbm.at[didx_v.at[j]])


def _gather_body(stid2_hbm, t_hbm, gath_hbm,
                 sidx4_v, rows_a, rows_b, sem, sem2, sem3):
    c = lax.axis_index("c")
    s = lax.axis_index("s")
    w = c * SC_NS + s
    my_slot0 = w * SPT
    pltpu.sync_copy(stid2_hbm.at[pl.ds(w * (SPT // GCH), SPT // GCH)],
                    sidx4_v)
    rows = (rows_a, rows_b)
    osems = (sem2, sem3)
    handles = [None, None]
    for g in range(SPT // GCH):
        b = g % 2
        if handles[b] is not None:
            handles[b].wait()
        pltpu.async_copy(t_hbm.at[sidx4_v.at[g]], rows[b], sem).wait()
        handles[b] = pltpu.async_copy(
            rows[b], gath_hbm.at[pl.ds(my_slot0 + g * GCH, GCH)], osems[b])
    handles[0].wait()
    handles[1].wait()


def _combine_body(y_hbm, x_hbm, d0_hbm, d1_hbm, out_hbm,
                  d0_v, d1_v, a_v, b_v, o_v, sem):
    c = lax.axis_index("c")
    s = lax.axis_index("s")
    w = s * SC_NC + c
    tok0 = w * TPT
    r0 = w * (TPT // CCH)
    pltpu.sync_copy(d0_hbm.at[pl.ds(r0, TPT // CCH)], d0_v)
    pltpu.sync_copy(d1_hbm.at[pl.ds(r0, TPT // CCH)], d1_v)
    for ch in range(TPT // CCH):
        base = tok0 + ch * CCH
        pltpu.async_copy(y_hbm.at[d0_v.at[ch]], a_v, sem).wait()
        pltpu.async_copy(y_hbm.at[d1_v.at[ch]], b_v, sem).wait()
        pltpu.sync_copy(x_hbm.at[pl.ds(base, CCH)], o_v)
        for r in range(CCH):
            def add_vec(j, _, r=r):
                sl = pl.ds(j * 16, 16)
                o_v[r, sl] = o_v[r, sl] + a_v[r, sl] + b_v[r, sl]
                return 0
            lax.fori_loop(0, D // 16, add_vec, 0, unroll=8)
        pltpu.sync_copy(o_v, out_hbm.at[pl.ds(base, CCH)])


def _ffn_body(te_ref, g_ref, sw_ref, wg_ref, wu_ref, wd_ref,
              bg_ref, bu_ref, bd_ref, y_ref):
    ti = pl.program_id(0)
    e = jnp.maximum(te_ref[ti], 0)

    @pl.when(te_ref[ti] >= 0)
    def _():
        t = g_ref[...].astype(jnp.bfloat16)
        gate = jnp.dot(t, wg_ref[e], preferred_element_type=jnp.float32)
        gate = gate + bg_ref[e]
        up = jnp.dot(t, wu_ref[e], preferred_element_type=jnp.float32)
        up = up + bu_ref[e]
        gate = jnp.minimum(gate, LIMIT)
        up = jnp.clip(up, -LIMIT, LIMIT)
        glu = gate * jax.nn.sigmoid(gate * ALPHA)
        act = ((up + 1.0) * glu).astype(jnp.bfloat16)
        y = jnp.dot(act, wd_ref[e], preferred_element_type=jnp.float32)
        y_ref[...] = (y + bd_ref[e]) * sw_ref[...]


@jax.jit
def _mlpblock(x, norm_w, router_w, router_b, w_gate_up, b_gate_up, w_down,
              b_down):
    rwt = router_w.T
    rb = router_b.reshape(1, E)
    bg = b_gate_up[:, 0::2].reshape(E, 1, F)
    bu = b_gate_up[:, 1::2].reshape(E, 1, F)
    wd = w_down.astype(jnp.bfloat16)
    bd = b_down.reshape(E, 1, D)
    nw = norm_w.reshape(1, D)

    wgu_i32 = jax.lax.bitcast_convert_type(
        w_gate_up.astype(jnp.bfloat16).reshape(E, D, F, 2), jnp.int32)
    wg, wu = pl.pallas_call(
        _deint_body,
        grid=(E,),
        in_specs=[pl.BlockSpec((1, D, F), lambda e_: (e_, 0, 0))],
        out_specs=[pl.BlockSpec((1, D, F), lambda e_: (e_, 0, 0)),
                   pl.BlockSpec((1, D, F), lambda e_: (e_, 0, 0))],
        out_shape=[jax.ShapeDtypeStruct((E, D, F), jnp.bfloat16),
                   jax.ShapeDtypeStruct((E, D, F), jnp.bfloat16)],
    )(wgu_i32)

    t, dest, wp, te = pl.pallas_call(
        _router_body,
        grid=(1,),
        in_specs=[
            pl.BlockSpec((T, D), lambda i: (0, 0)),
            pl.BlockSpec((1, D), lambda i: (0, 0)),
            pl.BlockSpec((D, E), lambda i: (0, 0)),
            pl.BlockSpec((1, E), lambda i: (0, 0)),
        ],
        out_specs=[
            pl.BlockSpec((T, D), lambda i: (0, 0)),
            pl.BlockSpec((T, 2), lambda i: (0, 0)),
            pl.BlockSpec((T, 2), lambda i: (0, 0)),
            pl.BlockSpec((64, 1), lambda i: (0, 0)),
        ],
        out_shape=[
            jax.ShapeDtypeStruct((T, D), jnp.float32),
            jax.ShapeDtypeStruct((T, 2), jnp.int32),
            jax.ShapeDtypeStruct((T, 2), jnp.float32),
            jax.ShapeDtypeStruct((64, 1), jnp.int32),
        ],
    )(x, nw, rwt, rb)

    dest2 = dest.reshape(2 * T // 128, 128)
    w2 = wp.reshape(2 * T // 128, 128)
    mesh = plsc.VectorSubcoreMesh(core_axis_name="c", subcore_axis_name="s")
    stid, sw = pl.kernel(
        _scatter_body,
        out_type=[
            jax.ShapeDtypeStruct((NALLOC,), jnp.int32),
            jax.ShapeDtypeStruct((NALLOC,), jnp.float32),
        ],
        mesh=mesh,
        scratch_types=[
            pltpu.VMEM((SPT,), jnp.int32),
            pltpu.VMEM((SPT,), jnp.float32),
            pltpu.VMEM((PPT // 128, 128), jnp.int32),
            pltpu.VMEM((PPT // 128, 128), jnp.int32),
            pltpu.VMEM((PPT // 128, 128), jnp.float32),
        ],
    )(dest2, w2)
    stid2 = stid[:NPAD].reshape(NPAD // GCH, GCH)
    slot_w = sw[:NPAD]
    gathered = pl.kernel(
        _gather_body,
        out_type=jax.ShapeDtypeStruct((NPAD, D), jnp.float32),
        mesh=plsc.VectorSubcoreMesh(core_axis_name="c",
                                    subcore_axis_name="s"),
        scratch_types=[
            pltpu.VMEM((SPT // GCH, GCH), jnp.int32),
            pltpu.VMEM((GCH, D), jnp.float32),
            pltpu.VMEM((GCH, D), jnp.float32),
            pltpu.SemaphoreType.DMA,
            pltpu.SemaphoreType.DMA,
            pltpu.SemaphoreType.DMA,
        ],
    )(stid2, t)

    te_flat = te.reshape(64)[:NTILES]
    y = pl.pallas_call(
        _ffn_body,
        grid_spec=pltpu.PrefetchScalarGridSpec(
            num_scalar_prefetch=1,
            grid=(NTILES,),
            in_specs=[
                pl.BlockSpec((BP, D), lambda ti, te: (ti, 0)),
                pl.BlockSpec((BP, 1), lambda ti, te: (ti, 0)),
                pl.BlockSpec((E, D, F), lambda ti, te: (0, 0, 0)),
                pl.BlockSpec((E, D, F), lambda ti, te: (0, 0, 0)),
                pl.BlockSpec((E, F, D), lambda ti, te: (0, 0, 0)),
                pl.BlockSpec((E, 1, F), lambda ti, te: (0, 0, 0)),
                pl.BlockSpec((E, 1, F), lambda ti, te: (0, 0, 0)),
                pl.BlockSpec((E, 1, D), lambda ti, te: (0, 0, 0)),
            ],
            out_specs=pl.BlockSpec((BP, D), lambda ti, te: (ti, 0)),
        ),
        out_shape=jax.ShapeDtypeStruct((NPAD, D), jnp.float32),
        compiler_params=pltpu.CompilerParams(
            dimension_semantics=("arbitrary",),
            vmem_limit_bytes=120 * 1024 * 1024,
        ),
    )(te_flat, gathered, slot_w.reshape(NPAD, 1), wg, wu, wd,
      bg, bu, bd)

    out = x + y[dest[:, 0]] + y[dest[:, 1]]
    return out
    out = pl.kernel(
        _combine_body,
        out_type=jax.ShapeDtypeStruct((T, D), jnp.float32),
        mesh=plsc.VectorSubcoreMesh(core_axis_name="c",
                                    subcore_axis_name="s"),
        scratch_types=[
            pltpu.VMEM((TPT // CCH, CCH), jnp.int32),
            pltpu.VMEM((TPT // CCH, CCH), jnp.int32),
            pltpu.VMEM((CCH, D), jnp.float32),
            pltpu.VMEM((CCH, D), jnp.float32),
            pltpu.VMEM((CCH, D), jnp.float32),
            pltpu.SemaphoreType.DMA,
        ],
    )(y, x, dest[:, 0].reshape(T // CCH, CCH),
      dest[:, 1].reshape(T // CCH, CCH))
    return out


def kernel(x, norm_w, router_w, router_b, w_gate_up, b_gate_up, w_down,
           b_down):
    return _mlpblock(x, norm_w, router_w, router_b, w_gate_up, b_gate_up,
                     w_down, b_down)


# speedup vs baseline: 1.0003x; 1.0003x over previous
import functools

import jax
import jax.numpy as jnp
from jax import lax
from jax.experimental import pallas as pl
from jax.experimental.pallas import tpu as pltpu
from jax.experimental.pallas import tpu_sc as plsc

T, D, F, E, TOP_K = 2048, 1024, 1024, 8, 2
LIMIT = 7.0
ALPHA = 1.702
EPS = 1e-5

BP = 128
NPAD = 5120
NTILES = NPAD // BP

SC_NC, SC_NS = 2, 16
HALF = NPAD // SC_NC
SPT = HALF // SC_NS
PPT = (2 * T) // SC_NS
NALLOC = NPAD + 16
GCH = 40
TPT = T // (SC_NC * SC_NS)
CCH = 16


def _deint_body(p_ref, wg_ref, wu_ref):
    v = p_ref[0]
    wg_ref[0] = jax.lax.bitcast_convert_type(
        v << 16, jnp.float32).astype(jnp.bfloat16)
    wu_ref[0] = jax.lax.bitcast_convert_type(
        v & jnp.int32(-65536), jnp.float32).astype(jnp.bfloat16)


def _router_body(x_ref, nw_ref, rwt_ref, rb_ref,
                 t_ref, dest_ref, wp_ref, te_ref):
    xx = x_ref[...]
    ms = jnp.mean(xx * xx, axis=-1, keepdims=True)
    t = xx * jax.lax.rsqrt(ms + EPS) * nw_ref[...]
    t_ref[...] = t
    g = jnp.dot(t, rwt_ref[...], preferred_element_type=jnp.float32)
    g = g + rb_ref[...]
    m = jnp.max(g, axis=-1, keepdims=True)
    eg = jnp.exp(g - m)
    p = eg / jnp.sum(eg, axis=-1, keepdims=True)
    eidx = jax.lax.broadcasted_iota(jnp.int32, p.shape, 1)
    v1 = jnp.max(p, axis=-1, keepdims=True)
    i1 = jnp.min(jnp.where(p >= v1, eidx, E), axis=-1, keepdims=True)
    p2 = jnp.where(eidx == i1, -jnp.inf, p)
    v2 = jnp.max(p2, axis=-1, keepdims=True)
    i2 = jnp.min(jnp.where(p2 >= v2, eidx, E), axis=-1, keepdims=True)
    s = v1 + v2
    wp_ref[...] = jnp.concatenate([v1 / s, v2 / s], axis=1)

    oh = ((eidx == i1) | (eidx == i2)).astype(jnp.bfloat16)
    ri = jax.lax.broadcasted_iota(jnp.int32, (T, T), 0)
    ci = jax.lax.broadcasted_iota(jnp.int32, (T, T), 1)
    tri = (ri > ci).astype(jnp.bfloat16)
    acc = jnp.dot(tri, oh, preferred_element_type=jnp.float32)
    cnt = jnp.sum(oh.astype(jnp.float32), axis=0, keepdims=True)
    cp = jnp.ceil(cnt / BP) * BP
    ei8 = jax.lax.broadcasted_iota(jnp.int32, (E, E), 0)
    ej8 = jax.lax.broadcasted_iota(jnp.int32, (E, E), 1)
    upper = (ei8 < ej8).astype(jnp.float32)
    starts = jnp.dot(cp, upper, preferred_element_type=jnp.float32)
    total = jnp.sum(cp, axis=-1, keepdims=True)

    def sel(ik, mat):
        return jnp.sum(jnp.where(eidx == ik, mat, 0.0), axis=-1,
                       keepdims=True)

    starts_b = jnp.broadcast_to(starts, (T, E))
    d0 = sel(i1, starts_b) + sel(i1, acc)
    d1 = sel(i2, starts_b) + sel(i2, acc)
    dest_ref[...] = jnp.concatenate([d0, d1], axis=1).astype(jnp.int32)

    pos = (jax.lax.broadcasted_iota(jnp.int32, (64, E), 0) * BP).astype(
        jnp.float32)
    n_le = jnp.sum((jnp.broadcast_to(starts, (64, E)) <= pos).astype(
        jnp.int32), axis=-1, keepdims=True)
    tile_e = n_le - 1
    te_ref[...] = jnp.where(pos[:, :1] < total, tile_e, -1)


def _scatter_body(dest_hbm, wflat_hbm, stid_hbm, sw_hbm,
                  zi_v, zf_v, didx_v, tid_v, wv_v):
    c = lax.axis_index("c")
    s = lax.axis_index("s")
    half = c * HALF
    my_slot0 = half + s * SPT

    for i in range(SPT // 16):
        zi_v[pl.ds(i * 16, 16)] = jnp.zeros((16,), jnp.int32)
        zf_v[pl.ds(i * 16, 16)] = jnp.zeros((16,), jnp.float32)
    pltpu.sync_copy(zi_v, stid_hbm.at[pl.ds(my_slot0, SPT)])
    pltpu.sync_copy(zf_v, sw_hbm.at[pl.ds(my_slot0, SPT)])
    plsc.subcore_barrier()

    p0 = s * PPT
    pltpu.sync_copy(dest_hbm.at[pl.ds(s * (PPT // 128), PPT // 128)],
                    didx_v)
    pltpu.sync_copy(wflat_hbm.at[pl.ds(s * (PPT // 128), PPT // 128)],
                    wv_v)
    dump = jnp.int32(NPAD) + 8 * c
    for j in range(PPT // 128):
        for i in range(128 // 16):
            d = didx_v[j, pl.ds(i * 16, 16)]
            keep = (d >= half) & (d < half + HALF)
            didx_v[j, pl.ds(i * 16, 16)] = jnp.where(keep, d, dump)
            pid = lax.iota(jnp.int32, 16) + (p0 + j * 128 + i * 16)
            tid_v[j, pl.ds(i * 16, 16)] = pid >> 1

    for j in range(PPT // 128):
        pltpu.sync_copy(tid_v.at[j], stid_hbm.at[didx_v.at[j]])
        pltpu.sync_copy(wv_v.at[j], sw_hbm.at[didx_v.at[j]])


def _gather_body(idx2_hbm, t_hbm, gath_hbm,
                 sidx4_v, rows_a, rows_b, sem, sem2, sem3, *, spt, gch):
    c = lax.axis_index("c")
    s = lax.axis_index("s")
    w = c * SC_NS + s
    my_slot0 = w * spt
    pltpu.sync_copy(idx2_hbm.at[pl.ds(w * (spt // gch), spt // gch)],
                    sidx4_v)
    rows = (rows_a, rows_b)
    osems = (sem2, sem3)
    handles = [None, None]
    for g in range(spt // gch):
        b = g % 2
        if handles[b] is not None:
            handles[b].wait()
        pltpu.async_copy(t_hbm.at[sidx4_v.at[g]], rows[b], sem).wait()
        handles[b] = pltpu.async_copy(
            rows[b], gath_hbm.at[pl.ds(my_slot0 + g * gch, gch)], osems[b])
    handles[0].wait()
    handles[1].wait()


def _add_body(x_ref, a_ref, b_ref, o_ref):
    o_ref[...] = x_ref[...] + a_ref[...] + b_ref[...]


def _ffn_body(te_ref, g_ref, sw_ref, wg_ref, wu_ref, wd_ref,
              bg_ref, bu_ref, bd_ref, y_ref):
    ti = pl.program_id(0)
    e = jnp.maximum(te_ref[ti], 0)

    @pl.when(te_ref[ti] >= 0)
    def _():
        t = g_ref[...].astype(jnp.bfloat16)
        gate = jnp.dot(t, wg_ref[e], preferred_element_type=jnp.float32)
        gate = gate + bg_ref[e]
        up = jnp.dot(t, wu_ref[e], preferred_element_type=jnp.float32)
        up = up + bu_ref[e]
        gate = jnp.minimum(gate, LIMIT)
        up = jnp.clip(up, -LIMIT, LIMIT)
        glu = gate * jax.nn.sigmoid(gate * ALPHA)
        act = ((up + 1.0) * glu).astype(jnp.bfloat16)
        y = jnp.dot(act, wd_ref[e], preferred_element_type=jnp.float32)
        y_ref[...] = (y + bd_ref[e]) * sw_ref[...]


@jax.jit
def _mlpblock(x, norm_w, router_w, router_b, w_gate_up, b_gate_up, w_down,
              b_down):
    rwt = router_w.T
    rb = router_b.reshape(1, E)
    bg = b_gate_up[:, 0::2].reshape(E, 1, F)
    bu = b_gate_up[:, 1::2].reshape(E, 1, F)
    wd = w_down.astype(jnp.bfloat16)
    bd = b_down.reshape(E, 1, D)
    nw = norm_w.reshape(1, D)

    wgu_i32 = jax.lax.bitcast_convert_type(
        w_gate_up.astype(jnp.bfloat16).reshape(E, D, F, 2), jnp.int32)
    wg, wu = pl.pallas_call(
        _deint_body,
        grid=(E,),
        in_specs=[pl.BlockSpec((1, D, F), lambda e_: (e_, 0, 0))],
        out_specs=[pl.BlockSpec((1, D, F), lambda e_: (e_, 0, 0)),
                   pl.BlockSpec((1, D, F), lambda e_: (e_, 0, 0))],
        out_shape=[jax.ShapeDtypeStruct((E, D, F), jnp.bfloat16),
                   jax.ShapeDtypeStruct((E, D, F), jnp.bfloat16)],
    )(wgu_i32)

    t, dest, wp, te = pl.pallas_call(
        _router_body,
        grid=(1,),
        in_specs=[
            pl.BlockSpec((T, D), lambda i: (0, 0)),
            pl.BlockSpec((1, D), lambda i: (0, 0)),
            pl.BlockSpec((D, E), lambda i: (0, 0)),
            pl.BlockSpec((1, E), lambda i: (0, 0)),
        ],
        out_specs=[
            pl.BlockSpec((T, D), lambda i: (0, 0)),
            pl.BlockSpec((T, 2), lambda i: (0, 0)),
            pl.BlockSpec((T, 2), lambda i: (0, 0)),
            pl.BlockSpec((64, 1), lambda i: (0, 0)),
        ],
        out_shape=[
            jax.ShapeDtypeStruct((T, D), jnp.float32),
            jax.ShapeDtypeStruct((T, 2), jnp.int32),
            jax.ShapeDtypeStruct((T, 2), jnp.float32),
            jax.ShapeDtypeStruct((64, 1), jnp.int32),
        ],
    )(x, nw, rwt, rb)

    dest2 = dest.reshape(2 * T // 128, 128)
    w2 = wp.reshape(2 * T // 128, 128)
    mesh = plsc.VectorSubcoreMesh(core_axis_name="c", subcore_axis_name="s")
    stid, sw = pl.kernel(
        _scatter_body,
        out_type=[
            jax.ShapeDtypeStruct((NALLOC,), jnp.int32),
            jax.ShapeDtypeStruct((NALLOC,), jnp.float32),
        ],
        mesh=mesh,
        scratch_types=[
            pltpu.VMEM((SPT,), jnp.int32),
            pltpu.VMEM((SPT,), jnp.float32),
            pltpu.VMEM((PPT // 128, 128), jnp.int32),
            pltpu.VMEM((PPT // 128, 128), jnp.int32),
            pltpu.VMEM((PPT // 128, 128), jnp.float32),
        ],
    )(dest2, w2)
    stid2 = stid[:NPAD].reshape(NPAD // GCH, GCH)
    slot_w = sw[:NPAD]
    gathered = pl.kernel(
        functools.partial(_gather_body, spt=SPT, gch=GCH),
        out_type=jax.ShapeDtypeStruct((NPAD, D), jnp.float32),
        mesh=plsc.VectorSubcoreMesh(core_axis_name="c",
                                    subcore_axis_name="s"),
        scratch_types=[
            pltpu.VMEM((SPT // GCH, GCH), jnp.int32),
            pltpu.VMEM((GCH, D), jnp.float32),
            pltpu.VMEM((GCH, D), jnp.float32),
            pltpu.SemaphoreType.DMA,
            pltpu.SemaphoreType.DMA,
            pltpu.SemaphoreType.DMA,
        ],
    )(stid2, t)

    te_flat = te.reshape(64)[:NTILES]
    y = pl.pallas_call(
        _ffn_body,
        grid_spec=pltpu.PrefetchScalarGridSpec(
            num_scalar_prefetch=1,
            grid=(NTILES,),
            in_specs=[
                pl.BlockSpec((BP, D), lambda ti, te: (ti, 0)),
                pl.BlockSpec((BP, 1), lambda ti, te: (ti, 0)),
                pl.BlockSpec((E, D, F), lambda ti, te: (0, 0, 0)),
                pl.BlockSpec((E, D, F), lambda ti, te: (0, 0, 0)),
                pl.BlockSpec((E, F, D), lambda ti, te: (0, 0, 0)),
                pl.BlockSpec((E, 1, F), lambda ti, te: (0, 0, 0)),
                pl.BlockSpec((E, 1, F), lambda ti, te: (0, 0, 0)),
                pl.BlockSpec((E, 1, D), lambda ti, te: (0, 0, 0)),
            ],
            out_specs=pl.BlockSpec((BP, D), lambda ti, te: (ti, 0)),
        ),
        out_shape=jax.ShapeDtypeStruct((NPAD, D), jnp.float32),
        compiler_params=pltpu.CompilerParams(
            dimension_semantics=("arbitrary",),
            vmem_limit_bytes=120 * 1024 * 1024,
        ),
    )(te_flat, gathered, slot_w.reshape(NPAD, 1), wg, wu, wd,
      bg, bu, bd)

    out = x + y[dest[:, 0]] + y[dest[:, 1]]
    return out
    CG = 32
    idxcat = jnp.concatenate([dest[:, 0], dest[:, 1]]).reshape(
        2 * T // CG, CG)
    yg = pl.kernel(
        functools.partial(_gather_body, spt=2 * T // 32, gch=CG),
        out_type=jax.ShapeDtypeStruct((2 * T, D), jnp.float32),
        mesh=plsc.VectorSubcoreMesh(core_axis_name="c",
                                    subcore_axis_name="s"),
        scratch_types=[
            pltpu.VMEM(((2 * T // 32) // CG, CG), jnp.int32),
            pltpu.VMEM((CG, D), jnp.float32),
            pltpu.VMEM((CG, D), jnp.float32),
            pltpu.SemaphoreType.DMA,
            pltpu.SemaphoreType.DMA,
            pltpu.SemaphoreType.DMA,
        ],
    )(idxcat, y)
    out = pl.pallas_call(
        _add_body,
        grid=(T // 256,),
        in_specs=[
            pl.BlockSpec((256, D), lambda i: (i, 0)),
            pl.BlockSpec((256, D), lambda i: (i, 0)),
            pl.BlockSpec((256, D), lambda i: (i, 0)),
        ],
        out_specs=pl.BlockSpec((256, D), lambda i: (i, 0)),
        out_shape=jax.ShapeDtypeStruct((T, D), jnp.float32),
    )(x, yg[:T], yg[T:])
    return out


def kernel(x, norm_w, router_w, router_b, w_gate_up, b_gate_up, w_down,
           b_down):
    return _mlpblock(x, norm_w, router_w, router_b, w_gate_up, b_gate_up,
                     w_down, b_down)
